# baseline (speedup 1.0000x reference)
import functools

import jax
import jax.numpy as jnp
from jax import lax
from jax.experimental import pallas as pl
from jax.experimental.pallas import tpu as pltpu
from jax.experimental.pallas import tpu_sc as plsc

_NC = 2
_NS = 16
_NW = _NC * _NS
_L = 16
_CHUNK = 128
_NB = _CHUNK // _L


@functools.lru_cache(maxsize=None)
def _make_emb(B: int, V: int, D: int):
  BPW = B // _NW
  NCH = BPW // _CHUNK
  mesh = plsc.VectorSubcoreMesh(
      core_axis_name="c", subcore_axis_name="s",
      num_cores=_NC, num_subcores=_NS,
  )

  @functools.partial(
      pl.kernel,
      out_type=jax.ShapeDtypeStruct((B * D,), jnp.float32),
      mesh=mesh,
      scratch_types=[
          pltpu.VMEM((V * D,), jnp.float32),
          pltpu.VMEM((_CHUNK,), jnp.int32),
          pltpu.VMEM((_CHUNK,), jnp.int32),
          pltpu.VMEM((_CHUNK * D,), jnp.float32),
          pltpu.VMEM((_CHUNK * D,), jnp.float32),
          pltpu.SemaphoreType.DMA,
          pltpu.SemaphoreType.DMA,
      ],
      compiler_params=pltpu.CompilerParams(
          use_tc_tiling_on_sc=False, needs_layout_passes=False,
      ),
  )
  def emb(idx_hbm, table_hbm, out_hbm, table_v, idx0, idx1, ob0, ob1,
          ssem0, ssem1):
    wid = lax.axis_index("s") * _NC + lax.axis_index("c")
    base0 = wid * BPW
    pltpu.sync_copy(table_hbm, table_v)
    lanes = lax.iota(jnp.int32, 16)
    idxs = (idx0, idx1)
    obs = (ob0, ob1)
    ssems = (ssem0, ssem1)

    @pl.loop(0, NCH // 2)
    def _(j):
      for s in range(2):
        base = base0 + (j * 2 + s) * _CHUNK

        @pl.when(j > 0)
        def _():
          pltpu.make_async_copy(
              obs[s], out_hbm.at[pl.ds(base * D, _CHUNK * D)], ssems[s]
          ).wait()

        pltpu.sync_copy(idx_hbm.at[pl.ds(base, _CHUNK)], idxs[s])
        for b in range(_NB):
          iv = idxs[s][pl.ds(b * _L, _L)]
          ga0 = iv * D
          sa0 = (lanes + b * _L) * D

          @pl.loop(0, D, init_carry=(ga0, sa0), unroll=20)
          def _(col, carry):
            ga, sa = carry
            plsc.store_scatter(obs[s], [sa], plsc.load_gather(table_v, [ga]))
            return ga + 1, sa + 1

        pltpu.async_copy(
            obs[s], out_hbm.at[pl.ds(base * D, _CHUNK * D)], ssems[s]
        )

    for s in range(2):
      last = base0 + (NCH - 2 + s) * _CHUNK
      pltpu.make_async_copy(
          obs[s], out_hbm.at[pl.ds(last * D, _CHUNK * D)], ssems[s]
      ).wait()

  return emb


def kernel(indices, table):
  B = indices.size
  V, D = table.shape
  idx = indices.reshape(B).astype(jnp.int32)
  grain = _NW * 2 * _CHUNK
  Bp = (B + grain - 1) // grain * grain
  if Bp != B:
    idx = jnp.pad(idx, (0, Bp - B))
  out = _make_emb(Bp, V, D)(idx, table.reshape(V * D))
  out = out.reshape(Bp, D)
  if Bp != B:
    out = out[:B]
  return out.reshape(indices.shape + (D,))
